# baseline (speedup 1.0000x reference)
import functools

import jax
import jax.numpy as jnp
from jax import lax
from jax.experimental import pallas as pl
from jax.experimental.pallas import tpu as pltpu
from jax.experimental.pallas import tpu_sc as plsc

D_MODEL = 1024
D_FF = 4096
D_NEURON = 128
D_HIDDEN = 256
K = 64
LN_EPS = 1e-5

_SQRT_HALF = 0.7071067811865476


def _ln(x, g, b):
    m = jnp.mean(x, axis=-1, keepdims=True)
    v = jnp.mean((x - m) ** 2, axis=-1, keepdims=True)
    return (x - m) * jax.lax.rsqrt(v + LN_EPS) * g + b


def _gelu(x):
    return x * 0.5 * (1.0 + jax.lax.erf(x * _SQRT_HALF))


def _mm_t(a, b):
    return jax.lax.dot_general(a, b, (((1,), (1,)), ((), ())),
                               preferred_element_type=jnp.float32)


def _to_key(s):
    i = jax.lax.bitcast_convert_type(s, jnp.int32)
    return jnp.where(i >= 0, i, i ^ jnp.int32(0x7FFFFFFF))


def _router_h_body(x_ref, g_ref, b_ref, w1_ref, h_ref):
    xn = _ln(x_ref[...], g_ref[...], b_ref[...])
    h_ref[...] = _gelu(_mm_t(xn, w1_ref[...]))


def _router_h(x_flat, g, b, w1, tm=512):
    t = x_flat.shape[0]
    return pl.pallas_call(
        _router_h_body,
        grid=(t // tm,),
        in_specs=[
            pl.BlockSpec((tm, D_MODEL), lambda i: (i, 0)),
            pl.BlockSpec((1, D_MODEL), lambda i: (0, 0)),
            pl.BlockSpec((1, D_MODEL), lambda i: (0, 0)),
            pl.BlockSpec((D_MODEL, D_MODEL), lambda i: (0, 0)),
        ],
        out_specs=pl.BlockSpec((tm, D_MODEL), lambda i: (i, 0)),
        out_shape=jax.ShapeDtypeStruct((t, D_MODEL), jnp.float32),
    )(x_flat, g, b, w1)


def _keys_mm_body(l_ref, w_ref, o_ref):
    o_ref[...] = _to_key(_mm_t(l_ref[...], w_ref[...]))


def _acts_mm_body(l_ref, w_ref, o_ref):
    o_ref[...] = _mm_t(l_ref[...], w_ref[...])


def _big_mm(body, lhs, w, out_dtype, tm=512, tn=1024):
    t = lhs.shape[0]
    return pl.pallas_call(
        body,
        grid=(t // tm, D_FF // tn),
        in_specs=[
            pl.BlockSpec((tm, D_MODEL), lambda i, j: (i, 0)),
            pl.BlockSpec((tn, D_MODEL), lambda i, j: (j, 0)),
        ],
        out_specs=pl.BlockSpec((tm, tn), lambda i, j: (i, j)),
        out_shape=jax.ShapeDtypeStruct((t, D_FF), out_dtype),
    )(lhs, w)


def _thresh_body(k_ref, th_ref):
    keys = k_ref[...]
    tm = keys.shape[0]
    lo = jnp.full((tm, 1), jnp.int32(-2147483648))
    hi = jnp.full((tm, 1), jnp.int32(2147483647))

    def it(_, lohi):
        lo, hi = lohi
        mid = (lo & hi) + jax.lax.shift_right_arithmetic(lo ^ hi, 1) \
            + ((lo ^ hi) & jnp.int32(1))
        cnt = jnp.sum((keys >= mid).astype(jnp.int32), axis=1, keepdims=True)
        ge = cnt >= K
        return jnp.where(ge, mid, lo), jnp.where(ge, hi, mid - 1)

    lo, hi = jax.lax.fori_loop(0, 32, it, (lo, hi))
    th_ref[...] = lo


def _thresh(keys, tm=256):
    t = keys.shape[0]
    return pl.pallas_call(
        _thresh_body,
        grid=(t // tm,),
        in_specs=[pl.BlockSpec((tm, D_FF), lambda i: (i, 0))],
        out_specs=pl.BlockSpec((tm, 1), lambda i: (i, 0)),
        out_shape=jax.ShapeDtypeStruct((t, 1), jnp.int32),
    )(keys)


_BT = 4
_IDXCAP = 96


def _vgather(vec16, idx16):
    return jax.lax.gather(
        vec16, idx16[:, None],
        dimension_numbers=jax.lax.GatherDimensionNumbers(
            offset_dims=(), collapsed_slice_dims=(0,), start_index_map=(0,)),
        slice_sizes=(1,), mode=jax.lax.GatherScatterMode.PROMISE_IN_BOUNDS)


def _vbcast(vec16, pos):
    return _vgather(vec16, jnp.full((16,), pos, jnp.int32))


def _sc_route(keys, theta, acts_full, neuron_vecs):
    t = keys.shape[0]
    info = plsc.get_sparse_core_info()
    nc, ns = info.num_cores, info.num_subcores
    nw = nc * ns
    tpw = t // nw
    nb = tpw // _BT
    mesh = plsc.VectorSubcoreMesh(core_axis_name="c", subcore_axis_name="s")

    @functools.partial(
        pl.kernel,
        out_type=(
            jax.ShapeDtypeStruct((t * K, D_NEURON), jnp.float32),
            jax.ShapeDtypeStruct((t * K,), jnp.float32),
        ),
        mesh=mesh,
        scratch_types=[
            pltpu.VMEM((2, _BT, D_FF), jnp.int32),
            pltpu.VMEM((2 * _BT * D_FF,), jnp.float32),
            pltpu.VMEM((tpw,), jnp.int32),
            pltpu.VMEM((_BT * 16 * 65,), jnp.int32),
            pltpu.VMEM((_BT * K,), jnp.int32),
            pltpu.VMEM((_BT * K, D_NEURON), jnp.float32),
            pltpu.VMEM((tpw * K,), jnp.float32),
            pltpu.SemaphoreType.DMA,
            pltpu.SemaphoreType.DMA,
            pltpu.SemaphoreType.DMA,
            pltpu.SemaphoreType.DMA,
            pltpu.SemaphoreType.DMA,
        ],
        compiler_params=pltpu.CompilerParams(needs_layout_passes=False),
    )
    def body(keys_hbm, theta_hbm, acts_hbm, nv_hbm, nvg_out, asel_out,
             kb, ab, thv, lbuf, idxv, nvb, aselv, sk0, sk1, sa0, sa1, sg):
        wid = lax.axis_index("s") * nc + lax.axis_index("c")
        base = wid * tpw
        sks = (sk0, sk1)
        sas = (sa0, sa1)
        lane = jax.lax.iota(jnp.int32, 16)
        lane65 = lane * jnp.int32(65)
        zeros16 = jnp.zeros((16,), jnp.int32)

        pltpu.sync_copy(theta_hbm.at[pl.ds(base, tpw)], thv)
        pltpu.async_copy(keys_hbm.at[pl.ds(base, _BT)], kb.at[0], sks[0])
        for b in range(_BT):
            pltpu.async_copy(acts_hbm.at[base + b],
                             ab.at[pl.ds(b * D_FF, D_FF)], sas[0])

        def do_block(g, par):
            row0 = base + g * _BT
            @pl.when(g + 1 < nb)
            def _():
                nrow = row0 + _BT
                pltpu.async_copy(keys_hbm.at[pl.ds(nrow, _BT)],
                                 kb.at[1 - par], sks[1 - par])
                for b2 in range(_BT):
                    pltpu.async_copy(
                        acts_hbm.at[nrow + b2],
                        ab.at[pl.ds(((1 - par) * _BT + b2) * D_FF, D_FF)],
                        sas[1 - par])
            pltpu.make_async_copy(keys_hbm.at[pl.ds(row0, _BT)],
                                  kb.at[par], sks[par]).wait()
            for b2 in range(_BT):
                pltpu.make_async_copy(
                    acts_hbm.at[row0 + b2],
                    ab.at[pl.ds((par * _BT + b2) * D_FF, D_FF)],
                    sas[par]).wait()

            th_chunk = thv[pl.ds((g // 4) * 16, 16)]
            gathers = []
            for b in range(_BT):
                tloc = g * _BT + b
                thb = _vbcast(th_chunk, (g % 4) * _BT + b)
                lbase = b * 16 * 65
                laneslot = lane65 + lbase

                def scan(i, offv):
                    for u in range(4):
                        ii = i * 4 + u
                        kv = kb[par, b, pl.ds(ii * 16, 16)]
                        m = (kv >= thb) & (offv < K)
                        dest = laneslot + jnp.where(m, offv, jnp.int32(K))
                        plsc.store_scatter(lbuf, [dest], lane + ii * 16)
                        offv = offv + m.astype(jnp.int32)
                    return offv

                offv = lax.fori_loop(0, D_FF // 64, scan, zeros16)

                cum = plsc.cumsum(offv)
                cumx = cum - offv
                for jv in range(K // 16):
                    j = lane + jnp.int32(jv * 16)
                    lane_of = zeros16
                    for l in range(16):
                        lane_of = lane_of + (j >= _vbcast(cum, l)).astype(
                            jnp.int32)
                    rank = j - _vgather(cumx, lane_of)
                    sel = plsc.load_gather(
                        lbuf, [lane_of * jnp.int32(65) + rank + lbase])
                    idxv[pl.ds(b * K + jv * 16, 16)] = sel
                    av = plsc.load_gather(
                        ab, [sel + jnp.int32((par * _BT + b) * D_FF)])
                    aselv[pl.ds(tloc * K + jv * 16, 16)] = av

                gathers.append(pltpu.async_copy(
                    nv_hbm.at[idxv.at[pl.ds(b * K, K)]],
                    nvb.at[pl.ds(b * K, K)], sg))
            for cp in gathers:
                cp.wait()
            pltpu.sync_copy(nvb, nvg_out.at[pl.ds(row0 * K, _BT * K)])

        def outer(g2, _):
            do_block(g2 * 2, 0)
            do_block(g2 * 2 + 1, 1)
            return 0

        lax.fori_loop(0, nb // 2, outer, jnp.int32(0))
        pltpu.sync_copy(aselv, asel_out.at[pl.ds(base * K, tpw * K)])

    return body(keys, theta, acts_full, neuron_vecs)


def _phi_rho_body(nv_ref, acts_ref, a_ref, wl_ref, b1_ref, g1_ref, bl1_ref,
                  w2_ref, b2_ref, g2_ref, bl2_ref, rg_ref, rb_ref,
                  rw1_ref, rb1_ref, rw2_ref, rb2_ref, o_ref, *, tm):
    acts = _gelu(acts_ref[...])
    t = _mm_t(nv_ref[...], a_ref[...])
    t = t.reshape(tm, K, D_HIDDEN)
    t = t + acts[:, :, None] * wl_ref[...].reshape(1, 1, D_HIDDEN)
    t = t + b1_ref[...].reshape(1, 1, D_HIDDEN)
    t = _gelu(_ln(t, g1_ref[...], bl1_ref[...]))
    t = _mm_t(t.reshape(tm * K, D_HIDDEN), w2_ref[...]) + b2_ref[...]
    t = _ln(t, g2_ref[...], bl2_ref[...])
    agg = jnp.sum(t.reshape(tm, K, D_HIDDEN), axis=1)
    r = _ln(agg, rg_ref[...], rb_ref[...])
    r = _gelu(_mm_t(r, rw1_ref[...]) + rb1_ref[...])
    o_ref[...] = _mm_t(r, rw2_ref[...]) + rb2_ref[...]


def _phi_rho(nv_g, acts_sel, a_mat, w_last, b1, g1, bl1, w2, b2, g2, bl2,
             rg, rb, rw1, rb1, rw2, rb2, tm=128):
    t = acts_sel.shape[0]
    rep = lambda shape: pl.BlockSpec(shape, lambda i: tuple(0 for _ in shape))
    return pl.pallas_call(
        functools.partial(_phi_rho_body, tm=tm),
        grid=(t // tm,),
        in_specs=[
            pl.BlockSpec((tm * K, D_NEURON), lambda i: (i, 0)),
            pl.BlockSpec((tm, K), lambda i: (i, 0)),
            rep((D_HIDDEN, D_NEURON)),
            rep((1, D_HIDDEN)),
            rep((1, D_HIDDEN)),
            rep((1, D_HIDDEN)),
            rep((1, D_HIDDEN)),
            rep((D_HIDDEN, D_HIDDEN)),
            rep((1, D_HIDDEN)),
            rep((1, D_HIDDEN)),
            rep((1, D_HIDDEN)),
            rep((1, D_HIDDEN)),
            rep((1, D_HIDDEN)),
            rep((2 * D_HIDDEN, D_HIDDEN)),
            rep((1, 2 * D_HIDDEN)),
            rep((D_MODEL, 2 * D_HIDDEN)),
            rep((1, D_MODEL)),
        ],
        out_specs=pl.BlockSpec((tm, D_MODEL), lambda i: (i, 0)),
        out_shape=jax.ShapeDtypeStruct((t, D_MODEL), jnp.float32),
    )(nv_g, acts_sel, a_mat, w_last, b1, g1, bl1, w2, b2, g2, bl2,
      rg, rb, rw1, rb1, rw2, rb2)


def kernel(x, top_k, neuron_vecs, W_in, W_router_1, W_router_2,
           router_norm_g, router_norm_b,
           phi_w1, phi_b1, phi_ln1_g, phi_ln1_b,
           phi_w2, phi_b2, phi_ln2_g, phi_ln2_b,
           rho_ln_g, rho_ln_b, rho_w1, rho_b1, rho_w2, rho_b2):
    batch, seq, d_model = x.shape
    x_all = x.reshape(-1, d_model)
    r2 = lambda v: v.reshape(1, -1)
    a_mat = phi_w1[:, :D_NEURON]
    w_last = phi_w1[:, D_NEURON].reshape(1, -1)

    def half(x_flat):
        t = x_flat.shape[0]
        h = _router_h(x_flat, r2(router_norm_g), r2(router_norm_b),
                      W_router_1)
        keys = _big_mm(_keys_mm_body, h, W_router_2, jnp.int32)
        acts_full = _big_mm(_acts_mm_body, x_flat, W_in, jnp.float32)
        theta = _thresh(keys).reshape(t)
        nv_g, acts_sel_flat = _sc_route(keys, theta, acts_full, neuron_vecs)
        acts_sel = acts_sel_flat.reshape(t, K)
        return _phi_rho(nv_g, acts_sel, a_mat, w_last,
                        r2(phi_b1), r2(phi_ln1_g), r2(phi_ln1_b),
                        phi_w2, r2(phi_b2), r2(phi_ln2_g), r2(phi_ln2_b),
                        r2(rho_ln_g), r2(rho_ln_b),
                        rho_w1, r2(rho_b1), rho_w2, r2(rho_b2))

    tt = x_all.shape[0]
    nchunk = 8
    cs = tt // nchunk
    out = jnp.concatenate(
        [half(x_all[i * cs:(i + 1) * cs]) for i in range(nchunk)])
    return out.reshape(batch, seq, d_model)


# speedup vs baseline: 1.0132x; 1.0132x over previous
import functools

import jax
import jax.numpy as jnp
from jax import lax
from jax.experimental import pallas as pl
from jax.experimental.pallas import tpu as pltpu
from jax.experimental.pallas import tpu_sc as plsc

D_MODEL = 1024
D_FF = 4096
D_NEURON = 128
D_HIDDEN = 256
K = 64
LN_EPS = 1e-5

_SQRT_HALF = 0.7071067811865476


def _ln(x, g, b):
    m = jnp.mean(x, axis=-1, keepdims=True)
    v = jnp.mean((x - m) ** 2, axis=-1, keepdims=True)
    return (x - m) * jax.lax.rsqrt(v + LN_EPS) * g + b


def _gelu(x):
    return x * 0.5 * (1.0 + jax.lax.erf(x * _SQRT_HALF))


def _mm_t(a, b):
    return jax.lax.dot_general(a, b, (((1,), (1,)), ((), ())),
                               preferred_element_type=jnp.float32)


def _to_key(s):
    i = jax.lax.bitcast_convert_type(s, jnp.int32)
    return jnp.where(i >= 0, i, i ^ jnp.int32(0x7FFFFFFF))


def _router_h_body(x_ref, g_ref, b_ref, w1_ref, h_ref):
    xn = _ln(x_ref[...], g_ref[...], b_ref[...])
    h_ref[...] = _gelu(_mm_t(xn, w1_ref[...]))


def _router_h(x_flat, g, b, w1, tm=512):
    t = x_flat.shape[0]
    return pl.pallas_call(
        _router_h_body,
        grid=(t // tm,),
        in_specs=[
            pl.BlockSpec((tm, D_MODEL), lambda i: (i, 0)),
            pl.BlockSpec((1, D_MODEL), lambda i: (0, 0)),
            pl.BlockSpec((1, D_MODEL), lambda i: (0, 0)),
            pl.BlockSpec((D_MODEL, D_MODEL), lambda i: (0, 0)),
        ],
        out_specs=pl.BlockSpec((tm, D_MODEL), lambda i: (i, 0)),
        out_shape=jax.ShapeDtypeStruct((t, D_MODEL), jnp.float32),
    )(x_flat, g, b, w1)


def _keys_mm_body(l_ref, w_ref, o_ref):
    o_ref[...] = _to_key(_mm_t(l_ref[...], w_ref[...]))


def _acts_mm_body(l_ref, w_ref, o_ref):
    o_ref[...] = _mm_t(l_ref[...], w_ref[...])


def _big_mm(body, lhs, w, out_dtype, tm=512, tn=1024):
    t = lhs.shape[0]
    return pl.pallas_call(
        body,
        grid=(t // tm, D_FF // tn),
        in_specs=[
            pl.BlockSpec((tm, D_MODEL), lambda i, j: (i, 0)),
            pl.BlockSpec((tn, D_MODEL), lambda i, j: (j, 0)),
        ],
        out_specs=pl.BlockSpec((tm, tn), lambda i, j: (i, j)),
        out_shape=jax.ShapeDtypeStruct((t, D_FF), out_dtype),
    )(lhs, w)


def _thresh_body(k_ref, th_ref):
    keys = k_ref[...]
    tm = keys.shape[0]
    lo = jnp.full((tm, 1), jnp.int32(-2147483648))
    hi = jnp.full((tm, 1), jnp.int32(2147483647))

    def it(_, lohi):
        lo, hi = lohi
        mid = (lo & hi) + jax.lax.shift_right_arithmetic(lo ^ hi, 1) \
            + ((lo ^ hi) & jnp.int32(1))
        cnt = jnp.sum((keys >= mid).astype(jnp.int32), axis=1, keepdims=True)
        ge = cnt >= K
        return jnp.where(ge, mid, lo), jnp.where(ge, hi, mid - 1)

    lo, hi = jax.lax.fori_loop(0, 32, it, (lo, hi))
    th_ref[...] = lo


def _thresh(keys, tm=256):
    t = keys.shape[0]
    return pl.pallas_call(
        _thresh_body,
        grid=(t // tm,),
        in_specs=[pl.BlockSpec((tm, D_FF), lambda i: (i, 0))],
        out_specs=pl.BlockSpec((tm, 1), lambda i: (i, 0)),
        out_shape=jax.ShapeDtypeStruct((t, 1), jnp.int32),
    )(keys)


_BT = 2
_TPC = 16 // _BT
_IDXCAP = 96


def _vgather(vec16, idx16):
    return jax.lax.gather(
        vec16, idx16[:, None],
        dimension_numbers=jax.lax.GatherDimensionNumbers(
            offset_dims=(), collapsed_slice_dims=(0,), start_index_map=(0,)),
        slice_sizes=(1,), mode=jax.lax.GatherScatterMode.PROMISE_IN_BOUNDS)


def _vbcast(vec16, pos):
    return _vgather(vec16, jnp.full((16,), pos, jnp.int32))


def _sc_route(keys, theta, acts_full, neuron_vecs):
    t = keys.shape[0]
    info = plsc.get_sparse_core_info()
    nc, ns = info.num_cores, info.num_subcores
    nw = nc * ns
    tpw = t // nw
    nb = tpw // _BT
    mesh = plsc.VectorSubcoreMesh(core_axis_name="c", subcore_axis_name="s")

    @functools.partial(
        pl.kernel,
        out_type=(
            jax.ShapeDtypeStruct((t * K, D_NEURON), jnp.float32),
            jax.ShapeDtypeStruct((t * K,), jnp.float32),
        ),
        mesh=mesh,
        scratch_types=[
            pltpu.VMEM((2, _BT, D_FF), jnp.int32),
            pltpu.VMEM((2 * _BT * D_FF,), jnp.float32),
            pltpu.VMEM((tpw,), jnp.int32),
            pltpu.VMEM((_BT * 16 * 65,), jnp.int32),
            pltpu.VMEM((_BT * K,), jnp.int32),
            pltpu.VMEM((2, _BT * K, D_NEURON), jnp.float32),
            pltpu.VMEM((tpw * K,), jnp.float32),
            pltpu.SemaphoreType.DMA,
            pltpu.SemaphoreType.DMA,
            pltpu.SemaphoreType.DMA,
            pltpu.SemaphoreType.DMA,
            pltpu.SemaphoreType.DMA,
            pltpu.SemaphoreType.DMA,
            pltpu.SemaphoreType.DMA,
        ],
        compiler_params=pltpu.CompilerParams(needs_layout_passes=False),
    )
    def body(keys_hbm, theta_hbm, acts_hbm, nv_hbm, nvg_out, asel_out,
             kb, ab, thv, lbuf, idxv, nvb, aselv, sk0, sk1, sa0, sa1, sg,
             so0, so1):
        wid = lax.axis_index("s") * nc + lax.axis_index("c")
        base = wid * tpw
        sks = (sk0, sk1)
        sas = (sa0, sa1)
        sos = (so0, so1)
        lane = jax.lax.iota(jnp.int32, 16)
        lane65 = lane * jnp.int32(65)
        zeros16 = jnp.zeros((16,), jnp.int32)

        pltpu.sync_copy(theta_hbm.at[pl.ds(base, tpw)], thv)
        pltpu.async_copy(keys_hbm.at[pl.ds(base, _BT)], kb.at[0], sks[0])
        for b in range(_BT):
            pltpu.async_copy(acts_hbm.at[base + b],
                             ab.at[pl.ds(b * D_FF, D_FF)], sas[0])

        def do_block(g, par):
            row0 = base + g * _BT
            @pl.when(g + 1 < nb)
            def _():
                nrow = row0 + _BT
                pltpu.async_copy(keys_hbm.at[pl.ds(nrow, _BT)],
                                 kb.at[1 - par], sks[1 - par])
                for b2 in range(_BT):
                    pltpu.async_copy(
                        acts_hbm.at[nrow + b2],
                        ab.at[pl.ds(((1 - par) * _BT + b2) * D_FF, D_FF)],
                        sas[1 - par])
            pltpu.make_async_copy(keys_hbm.at[pl.ds(row0, _BT)],
                                  kb.at[par], sks[par]).wait()
            for b2 in range(_BT):
                pltpu.make_async_copy(
                    acts_hbm.at[row0 + b2],
                    ab.at[pl.ds((par * _BT + b2) * D_FF, D_FF)],
                    sas[par]).wait()

            @pl.when(g >= 2)
            def _():
                prow = base + (g - 2) * _BT
                pltpu.make_async_copy(
                    nvb.at[par], nvg_out.at[pl.ds(prow * K, _BT * K)],
                    sos[par]).wait()

            th_chunk = thv[pl.ds((g // _TPC) * 16, 16)]
            gathers = []
            for b in range(_BT):
                tloc = g * _BT + b
                thb = _vbcast(th_chunk, (g % _TPC) * _BT + b)
                lbase = b * 16 * 65
                laneslot = lane65 + lbase

                def scan(i, offv):
                    for u in range(4):
                        ii = i * 4 + u
                        kv = kb[par, b, pl.ds(ii * 16, 16)]
                        m = (kv >= thb) & (offv < K)
                        dest = laneslot + jnp.where(m, offv, jnp.int32(K))
                        plsc.store_scatter(lbuf, [dest], lane + ii * 16)
                        offv = offv + m.astype(jnp.int32)
                    return offv

                offv = lax.fori_loop(0, D_FF // 64, scan, zeros16)

                cum = plsc.cumsum(offv)
                cumx = cum - offv
                for jv in range(K // 16):
                    j = lane + jnp.int32(jv * 16)
                    lane_of = zeros16
                    for l in range(16):
                        lane_of = lane_of + (j >= _vbcast(cum, l)).astype(
                            jnp.int32)
                    rank = j - _vgather(cumx, lane_of)
                    sel = plsc.load_gather(
                        lbuf, [lane_of * jnp.int32(65) + rank + lbase])
                    idxv[pl.ds(b * K + jv * 16, 16)] = sel
                    av = plsc.load_gather(
                        ab, [sel + jnp.int32((par * _BT + b) * D_FF)])
                    aselv[pl.ds(tloc * K + jv * 16, 16)] = av

                gathers.append(pltpu.async_copy(
                    nv_hbm.at[idxv.at[pl.ds(b * K, K)]],
                    nvb.at[par, pl.ds(b * K, K)], sg))
            for cp in gathers:
                cp.wait()
            pltpu.async_copy(nvb.at[par],
                             nvg_out.at[pl.ds(row0 * K, _BT * K)], sos[par])

        def outer(g2, _):
            do_block(g2 * 2, 0)
            do_block(g2 * 2 + 1, 1)
            return 0

        lax.fori_loop(0, nb // 2, outer, jnp.int32(0))
        for gg in (nb - 2, nb - 1):
            pltpu.make_async_copy(
                nvb.at[gg % 2],
                nvg_out.at[pl.ds((base + gg * _BT) * K, _BT * K)],
                sos[gg % 2]).wait()
        pltpu.sync_copy(aselv, asel_out.at[pl.ds(base * K, tpw * K)])

    return body(keys, theta, acts_full, neuron_vecs)


def _phi_rho_body(nv_ref, acts_ref, a_ref, wl_ref, b1_ref, g1_ref, bl1_ref,
                  w2_ref, b2_ref, g2_ref, bl2_ref, rg_ref, rb_ref,
                  rw1_ref, rb1_ref, rw2_ref, rb2_ref, o_ref, *, tm):
    acts = _gelu(acts_ref[...])
    t = _mm_t(nv_ref[...], a_ref[...])
    t = t.reshape(tm, K, D_HIDDEN)
    t = t + acts[:, :, None] * wl_ref[...].reshape(1, 1, D_HIDDEN)
    t = t + b1_ref[...].reshape(1, 1, D_HIDDEN)
    t = _gelu(_ln(t, g1_ref[...], bl1_ref[...]))
    t = _mm_t(t.reshape(tm * K, D_HIDDEN), w2_ref[...]) + b2_ref[...]
    t = _ln(t, g2_ref[...], bl2_ref[...])
    agg = jnp.sum(t.reshape(tm, K, D_HIDDEN), axis=1)
    r = _ln(agg, rg_ref[...], rb_ref[...])
    r = _gelu(_mm_t(r, rw1_ref[...]) + rb1_ref[...])
    o_ref[...] = _mm_t(r, rw2_ref[...]) + rb2_ref[...]


def _phi_rho(nv_g, acts_sel, a_mat, w_last, b1, g1, bl1, w2, b2, g2, bl2,
             rg, rb, rw1, rb1, rw2, rb2, tm=128):
    t = acts_sel.shape[0]
    rep = lambda shape: pl.BlockSpec(shape, lambda i: tuple(0 for _ in shape))
    return pl.pallas_call(
        functools.partial(_phi_rho_body, tm=tm),
        grid=(t // tm,),
        in_specs=[
            pl.BlockSpec((tm * K, D_NEURON), lambda i: (i, 0)),
            pl.BlockSpec((tm, K), lambda i: (i, 0)),
            rep((D_HIDDEN, D_NEURON)),
            rep((1, D_HIDDEN)),
            rep((1, D_HIDDEN)),
            rep((1, D_HIDDEN)),
            rep((1, D_HIDDEN)),
            rep((D_HIDDEN, D_HIDDEN)),
            rep((1, D_HIDDEN)),
            rep((1, D_HIDDEN)),
            rep((1, D_HIDDEN)),
            rep((1, D_HIDDEN)),
            rep((1, D_HIDDEN)),
            rep((2 * D_HIDDEN, D_HIDDEN)),
            rep((1, 2 * D_HIDDEN)),
            rep((D_MODEL, 2 * D_HIDDEN)),
            rep((1, D_MODEL)),
        ],
        out_specs=pl.BlockSpec((tm, D_MODEL), lambda i: (i, 0)),
        out_shape=jax.ShapeDtypeStruct((t, D_MODEL), jnp.float32),
    )(nv_g, acts_sel, a_mat, w_last, b1, g1, bl1, w2, b2, g2, bl2,
      rg, rb, rw1, rb1, rw2, rb2)


def kernel(x, top_k, neuron_vecs, W_in, W_router_1, W_router_2,
           router_norm_g, router_norm_b,
           phi_w1, phi_b1, phi_ln1_g, phi_ln1_b,
           phi_w2, phi_b2, phi_ln2_g, phi_ln2_b,
           rho_ln_g, rho_ln_b, rho_w1, rho_b1, rho_w2, rho_b2):
    batch, seq, d_model = x.shape
    x_all = x.reshape(-1, d_model)
    r2 = lambda v: v.reshape(1, -1)
    a_mat = phi_w1[:, :D_NEURON]
    w_last = phi_w1[:, D_NEURON].reshape(1, -1)

    def half(x_flat):
        t = x_flat.shape[0]
        h = _router_h(x_flat, r2(router_norm_g), r2(router_norm_b),
                      W_router_1)
        keys = _big_mm(_keys_mm_body, h, W_router_2, jnp.int32)
        acts_full = _big_mm(_acts_mm_body, x_flat, W_in, jnp.float32)
        theta = _thresh(keys).reshape(t)
        nv_g, acts_sel_flat = _sc_route(keys, theta, acts_full, neuron_vecs)
        acts_sel = acts_sel_flat.reshape(t, K)
        return _phi_rho(nv_g, acts_sel, a_mat, w_last,
                        r2(phi_b1), r2(phi_ln1_g), r2(phi_ln1_b),
                        phi_w2, r2(phi_b2), r2(phi_ln2_g), r2(phi_ln2_b),
                        r2(rho_ln_g), r2(rho_ln_b),
                        rho_w1, r2(rho_b1), rho_w2, r2(rho_b2))

    tt = x_all.shape[0]
    nchunk = 4
    cs = tt // nchunk
    out = jnp.concatenate(
        [half(x_all[i * cs:(i + 1) * cs]) for i in range(nchunk)])
    return out.reshape(batch, seq, d_model)
